# baseline (speedup 1.0000x reference)
import functools

import jax
import jax.numpy as jnp
from jax import lax
from jax.experimental import pallas as pl
from jax.experimental.pallas import tpu as pltpu
from jax.experimental.pallas import tpu_sc as plsc

B, T, D = 64, 2048, 64
NC, NS = 2, 16
NW = NC * NS
TW = T // NW
BC = 8
NCH = B // BC
LANES = 16


def _run(idx_hbm, tok_hbm, pos_hbm, out_hbm, idx_v, pos_v, rows_v, sem):
    wid = lax.axis_index("s") * NC + lax.axis_index("c")
    t0 = wid * TW
    pltpu.sync_copy(idx_hbm.at[:, pl.ds(t0, TW)], idx_v)
    pltpu.sync_copy(pos_hbm.at[pl.ds(t0, TW), :], pos_v)
    for c in range(NCH):
        b0 = c * BC
        handles = [
            pltpu.async_copy(tok_hbm.at[idx_v.at[b0 + j]], rows_v.at[j], sem)
            for j in range(BC)
        ]
        for h in handles:
            h.wait()

        def add_pos(tt, carry):
            for q in range(D // LANES):
                v = pos_v[tt, pl.ds(q * LANES, LANES)]
                for j in range(BC):
                    plsc.addupdate(rows_v.at[j, tt, pl.ds(q * LANES, LANES)], v)
            return carry

        lax.fori_loop(0, TW, add_pos, 0)
        pltpu.sync_copy(rows_v, out_hbm.at[pl.ds(b0, BC), pl.ds(t0, TW), :])


def kernel(idx, token_table, position_table):
    mesh = plsc.VectorSubcoreMesh(core_axis_name="c", subcore_axis_name="s")
    run = functools.partial(
        pl.kernel,
        out_type=jax.ShapeDtypeStruct((B, T, D), jnp.float32),
        mesh=mesh,
        compiler_params=pltpu.CompilerParams(use_tc_tiling_on_sc=False),
        scratch_types=[
            pltpu.VMEM((B, TW), jnp.int32),
            pltpu.VMEM((TW, D), jnp.float32),
            pltpu.VMEM((BC, TW, D), jnp.float32),
            pltpu.SemaphoreType.DMA,
        ],
    )(_run)
    return run(idx.astype(jnp.int32), token_table, position_table)
